# speedup vs baseline: 4.3516x; 4.3516x over previous
import functools

import jax
import jax.numpy as jnp
from jax import lax
from jax.experimental import pallas as pl
from jax.experimental.pallas import tpu as pltpu
from jax.experimental.pallas import tpu_sc as plsc

N = 10000
D = 128
NC = 2
NS = 16
L = 16
CHUNK = 128
NP = 10240
RPT = NP // NS

_BCAST_DNUMS = lax.GatherDimensionNumbers(
    offset_dims=(), collapsed_slice_dims=(0,), start_index_map=(0,))


def _bcast_lane(v16, j):
    idx = jnp.full((L,), j, dtype=jnp.int32)
    return lax.gather(v16, idx[:, None], _BCAST_DNUMS, slice_sizes=(1,),
                      mode=lax.GatherScatterMode.PROMISE_IN_BOUNDS)


def _make_spmm(nchunk):
    mesh = plsc.VectorSubcoreMesh(
        core_axis_name="c", subcore_axis_name="s", num_cores=NC,
        num_subcores=NS)

    @functools.partial(
        pl.kernel,
        out_type=jax.ShapeDtypeStruct((NC, NP, D), jnp.float32),
        mesh=mesh,
        scratch_types=[
            pltpu.VMEM((nchunk, CHUNK), jnp.int32),
            pltpu.VMEM((nchunk, CHUNK), jnp.int32),
            pltpu.VMEM((nchunk, CHUNK), jnp.float32),
            pltpu.VMEM((CHUNK, D), jnp.float32),
            pltpu.VMEM_SHARED((NP, D), jnp.float32),
            pltpu.SemaphoreType.DMA,
        ],
    )
    def spmm(t1, colsi, rowsi, valsi, out, col_buf, row_buf, val_buf,
             gbuf, acc, sem):
        c = lax.axis_index("c")
        s = lax.axis_index("s")

        pltpu.sync_copy(colsi.at[c, s], col_buf)
        pltpu.sync_copy(rowsi.at[c, s], row_buf)
        pltpu.sync_copy(valsi.at[c, s], val_buf)

        zero16 = jnp.zeros((L,), jnp.float32)

        def zrow(r, carry):
            for q in range(D // L):
                gbuf[r, pl.ds(q * L, L)] = zero16
            return carry

        lax.fori_loop(0, CHUNK, zrow, 0)
        for k in range(RPT // CHUNK):
            pltpu.sync_copy(gbuf, acc.at[pl.ds(s * RPT + k * CHUNK, CHUNK)])
        plsc.subcore_barrier()

        def chunk_body(j, carry):
            pltpu.async_copy(t1.at[col_buf.at[j]], gbuf, sem).wait()

            def grp(g, carry2):
                v16 = val_buf[j, pl.ds(g * L, L)]
                for jj in range(L):
                    b = _bcast_lane(v16, jj)
                    e = g * L + jj
                    for q in range(D // L):
                        gbuf[e, pl.ds(q * L, L)] = (
                            gbuf[e, pl.ds(q * L, L)] * b)
                return carry2

            lax.fori_loop(0, CHUNK // L, grp, 0)
            pltpu.sync_copy(gbuf, acc.at[row_buf.at[j]], add=True)
            return carry

        lax.fori_loop(0, nchunk, chunk_body, 0)
        plsc.subcore_barrier()

        for k in range(RPT // CHUNK):
            pltpu.sync_copy(acc.at[pl.ds(s * RPT + k * CHUNK, CHUNK)],
                            out.at[c, pl.ds(s * RPT + k * CHUNK, CHUNK)])

    return spmm


def _combine_body(p_ref, t2_ref, th_ref, h_ref, h2_ref):
    ssum = p_ref[0] + p_ref[1]
    h = 2.0 * ssum - t2_ref[...]
    h_ref[...] = h
    h2_ref[...] = h * th_ref[...]


def kernel(T_n_1, T_n_2, edge_index, edge_vals, theta):
    E = edge_vals.shape[0]
    ept = -(-E // (NC * NS * CHUNK)) * CHUNK
    nchunk = ept // CHUNK
    EP = ept * NC * NS
    pad = EP - E

    col = jnp.concatenate(
        [edge_index[1], jnp.zeros((pad,), jnp.int32)]).reshape(
            NC, NS, nchunk, CHUNK)
    row = jnp.concatenate(
        [edge_index[0], jnp.zeros((pad,), jnp.int32)]).reshape(
            NC, NS, nchunk, CHUNK)
    val = jnp.concatenate(
        [edge_vals, jnp.zeros((pad,), jnp.float32)]).reshape(
            NC, NS, nchunk, CHUNK)

    partials = _make_spmm(nchunk)(T_n_1, col, row, val)

    R = 400
    th_b = jnp.broadcast_to(theta.reshape(1, 1), (1, D))
    H, H2 = pl.pallas_call(
        _combine_body,
        grid=(N // R,),
        in_specs=[
            pl.BlockSpec((NC, R, D), lambda i: (0, i, 0)),
            pl.BlockSpec((R, D), lambda i: (i, 0)),
            pl.BlockSpec((1, D), lambda i: (0, 0)),
        ],
        out_specs=[
            pl.BlockSpec((R, D), lambda i: (i, 0)),
            pl.BlockSpec((R, D), lambda i: (i, 0)),
        ],
        out_shape=[jax.ShapeDtypeStruct((N, D), jnp.float32)] * 2,
    )(partials, T_n_2, th_b)
    return (H, H2)
